# Initial kernel scaffold; baseline (speedup 1.0000x reference)
import jax
import jax.numpy as jnp
from jax.experimental import pallas as pl


def kernel(x, edge_index, W_in, b_in, gat1_W, gat1_att_src, gat1_att_dst, gat1_bias, gat2_W, gat2_att_src, gat2_att_dst, gat2_bias, fa_W1, fa_b1, fa_W2, fa_b2, cls_W, cls_b):
    raise NotImplementedError("write your pallas kernel here")


# speedup vs baseline: 1.1192x; 1.1192x over previous
import functools
import jax
import jax.numpy as jnp
from jax import lax
from jax.experimental import pallas as pl
from jax.experimental.pallas import tpu as pltpu
from jax.experimental.pallas import tpu_sc as plsc

WN = 256
BE = 256
CH = 128


def _group_mat(C, H, F):
    jr = lax.broadcasted_iota(jnp.int32, (C, H), 0) // F
    hc = lax.broadcasted_iota(jnp.int32, (C, H), 1)
    return jnp.where(jr == hc, 1.0, 0.0).astype(jnp.float32)


def _expand_mat(H, F):
    hr = lax.broadcasted_iota(jnp.int32, (H, H * F), 0)
    jc = lax.broadcasted_iota(jnp.int32, (H, H * F), 1) // F
    return jnp.where(hr == jc, 1.0, 0.0).astype(jnp.float32)


def _dense(mat, W, bias, attsF, attdF, H, F):
    R, Cin = mat.shape
    C = W.shape[1]
    grid = R // 128

    def body(h_ref, w_ref, b_ref, as_ref, ad_ref, T_ref, s_ref, d_ref, m_ref):
        b = pl.program_id(0)
        t = jnp.dot(h_ref[...], w_ref[...], preferred_element_type=jnp.float32)
        t = t + b_ref[...]
        T_ref[...] = t
        G = _group_mat(C, H, F)
        s = jnp.dot(t * as_ref[...], G, preferred_element_type=jnp.float32)
        d = jnp.dot(t * ad_ref[...], G, preferred_element_type=jnp.float32)
        s_ref[...] = s
        d_ref[...] = d
        bm = jnp.max(s, axis=0, keepdims=True)

        @pl.when(b == 0)
        def _():
            m_ref[...] = bm

        @pl.when(b > 0)
        def _():
            m_ref[...] = jnp.maximum(m_ref[...], bm)

    return pl.pallas_call(
        body,
        grid=(grid,),
        in_specs=[
            pl.BlockSpec((128, Cin), lambda b: (b, 0)),
            pl.BlockSpec((Cin, C), lambda b: (0, 0)),
            pl.BlockSpec((1, C), lambda b: (0, 0)),
            pl.BlockSpec((1, C), lambda b: (0, 0)),
            pl.BlockSpec((1, C), lambda b: (0, 0)),
        ],
        out_specs=[
            pl.BlockSpec((128, C), lambda b: (b, 0)),
            pl.BlockSpec((128, H), lambda b: (b, 0)),
            pl.BlockSpec((128, H), lambda b: (b, 0)),
            pl.BlockSpec((1, H), lambda b: (0, 0)),
        ],
        out_shape=[
            jax.ShapeDtypeStruct((R, C), jnp.float32),
            jax.ShapeDtypeStruct((R, H), jnp.float32),
            jax.ShapeDtypeStruct((R, H), jnp.float32),
            jax.ShapeDtypeStruct((1, H), jnp.float32),
        ],
    )(mat, W, bias, attsF, attdF)


def _sc_gather(table, idx, E3, D):
    info = plsc.get_sparse_core_info()
    NC, NS = info.num_cores, info.num_subcores
    NWK = NC * NS
    RPW = E3 // NWK
    TIT = RPW // CH
    mesh = plsc.VectorSubcoreMesh(core_axis_name="c", subcore_axis_name="s")

    @functools.partial(
        pl.kernel,
        mesh=mesh,
        out_type=jax.ShapeDtypeStruct((E3, D), jnp.float32),
        scratch_types=[
            pltpu.VMEM((CH,), jnp.int32),
            pltpu.VMEM((CH, D), jnp.float32),
            pltpu.SemaphoreType.DMA,
        ],
    )
    def k(tab_hbm, idx_hbm, out_hbm, idx_v, rows_v, sem):
        wid = lax.axis_index("s") * NC + lax.axis_index("c")
        base = wid * RPW

        def body(t, carry):
            b0 = base + t * CH
            pltpu.sync_copy(idx_hbm.at[pl.ds(b0, CH)], idx_v)
            pltpu.async_copy(tab_hbm.at[idx_v], rows_v, sem).wait()
            pltpu.sync_copy(rows_v, out_hbm.at[pl.ds(b0, CH)])
            return carry

        lax.fori_loop(0, TIT, body, 0)

    return k(table, idx)


def _edge(R, adstw, amax, dst3d, win_id, first, H, F, NPAD):
    D = R.shape[1]
    HF = H * F
    NBLK = R.shape[0] // BE

    def body(wi_ref, fl_ref, r_ref, aw_ref, am_ref, dst_ref, o_ref):
        b = pl.program_id(0)
        g = r_ref[...]
        feats = g[:, :HF]
        asrc = g[:, HF:HF + H]
        local = dst_ref[0, 0, :] - wi_ref[b] * WN
        wrow = lax.broadcasted_iota(jnp.int32, (WN, BE), 0)
        oh = jnp.where(wrow == local[None, :], 1.0, 0.0).astype(jnp.float32)
        ohT = jnp.where(
            lax.broadcasted_iota(jnp.int32, (BE, WN), 1) == local[:, None],
            1.0, 0.0).astype(jnp.float32)
        adst_e = jnp.dot(ohT, aw_ref[...], preferred_element_type=jnp.float32)
        am = am_ref[...]
        a = asrc + adst_e
        a = jnp.where(a > 0, a, 0.2 * a)
        c = jnp.maximum(am + adst_e, 0.0)
        p = jnp.exp(a - c)
        pfull = jnp.dot(p, _expand_mat(H, F), preferred_element_type=jnp.float32)
        msg = jnp.concatenate(
            [feats * pfull, p, jnp.zeros((BE, D - HF - H), jnp.float32)], axis=1)
        contrib = jnp.dot(oh, msg, preferred_element_type=jnp.float32)

        @pl.when(fl_ref[b] == 1)
        def _():
            o_ref[...] = contrib

        @pl.when(fl_ref[b] == 0)
        def _():
            o_ref[...] = o_ref[...] + contrib

    grid_spec = pltpu.PrefetchScalarGridSpec(
        num_scalar_prefetch=2,
        grid=(NBLK,),
        in_specs=[
            pl.BlockSpec((BE, D), lambda b, wi, fl: (b, 0)),
            pl.BlockSpec((WN, H), lambda b, wi, fl: (wi[b], 0)),
            pl.BlockSpec((1, H), lambda b, wi, fl: (0, 0)),
            pl.BlockSpec((1, 1, BE), lambda b, wi, fl: (b, 0, 0)),
        ],
        out_specs=pl.BlockSpec((WN, D), lambda b, wi, fl: (wi[b], 0)),
    )
    return pl.pallas_call(
        body,
        grid_spec=grid_spec,
        out_shape=jax.ShapeDtypeStruct((NPAD, D), jnp.float32),
    )(win_id, first, R, adstw, amax, dst3d)


def _norm(agg, bias, H, F):
    D = agg.shape[1]
    HF = H * F
    grid = agg.shape[0] // 128

    def body(a_ref, b_ref, o_ref):
        a = a_ref[...]
        d = a[:, HF:HF + H] + 1e-16
        dfull = jnp.dot(d, _expand_mat(H, F), preferred_element_type=jnp.float32)
        o_ref[...] = jnp.maximum(a[:, :HF] / dfull + b_ref[...], 0.0)

    return pl.pallas_call(
        body,
        grid=(grid,),
        in_specs=[
            pl.BlockSpec((128, D), lambda b: (b, 0)),
            pl.BlockSpec((1, HF), lambda b: (0, 0)),
        ],
        out_specs=pl.BlockSpec((128, HF), lambda b: (b, 0)),
        out_shape=jax.ShapeDtypeStruct((agg.shape[0], HF), jnp.float32),
    )(agg, bias)


def _head(h2p, faW1, fab1, faW2, fab2, clsW, clsb, NF):
    R = h2p.shape[0]
    grid = R // 128

    def body(h_ref, w1_ref, b1_ref, w2_ref, b2_ref, cw_ref, cb_ref,
             sc_ref, acc_ref, out_ref):
        b = pl.program_id(0)
        h = h_ref[...]
        a1 = jnp.maximum(
            jnp.dot(h, w1_ref[...], preferred_element_type=jnp.float32)
            + b1_ref[...], 0.0)
        z = jnp.dot(a1, w2_ref[...], preferred_element_type=jnp.float32) + b2_ref[...]
        sc_ref[...] = jax.nn.sigmoid(z)
        s = jnp.sum(h, axis=0, keepdims=True)

        @pl.when(b == 0)
        def _():
            acc_ref[...] = s

        @pl.when(b > 0)
        def _():
            acc_ref[...] = acc_ref[...] + s

        @pl.when(b == grid - 1)
        def _():
            out_ref[...] = jnp.dot(
                acc_ref[...] / NF, cw_ref[...],
                preferred_element_type=jnp.float32) + cb_ref[...]

    return pl.pallas_call(
        body,
        grid=(grid,),
        in_specs=[
            pl.BlockSpec((128, 64), lambda b: (b, 0)),
            pl.BlockSpec((64, 64), lambda b: (0, 0)),
            pl.BlockSpec((1, 64), lambda b: (0, 0)),
            pl.BlockSpec((64, 1), lambda b: (0, 0)),
            pl.BlockSpec((1, 1), lambda b: (0, 0)),
            pl.BlockSpec((64, 10), lambda b: (0, 0)),
            pl.BlockSpec((1, 10), lambda b: (0, 0)),
        ],
        out_specs=[
            pl.BlockSpec((128, 1), lambda b: (b, 0)),
            pl.BlockSpec((1, 64), lambda b: (0, 0)),
            pl.BlockSpec((1, 10), lambda b: (0, 0)),
        ],
        out_shape=[
            jax.ShapeDtypeStruct((R, 1), jnp.float32),
            jax.ShapeDtypeStruct((1, 64), jnp.float32),
            jax.ShapeDtypeStruct((1, 10), jnp.float32),
        ],
    )(h2p, faW1, fab1, faW2, fab2, clsW, clsb)


def kernel(x, edge_index, W_in, b_in, gat1_W, gat1_att_src, gat1_att_dst,
           gat1_bias, gat2_W, gat2_att_src, gat2_att_dst, gat2_bias,
           fa_W1, fa_b1, fa_W2, fa_b2, cls_W, cls_b):
    N = x.shape[0]
    E = edge_index.shape[1]
    NWIN = (N + WN - 1) // WN
    NPAD = NWIN * WN
    NB = ((N + 127) // 128) * 128
    E2 = E + N
    E3 = ((E2 + NWIN * (BE - 1) + 4095) // 4096) * 4096
    NBLK = E3 // BE

    loop = jnp.arange(N, dtype=edge_index.dtype)
    src2 = jnp.concatenate([edge_index[0], loop])
    dst2 = jnp.concatenate([edge_index[1], loop])
    sdst, ssrc = lax.sort((dst2, src2), num_keys=1)
    bnd = jnp.searchsorted(sdst, jnp.arange(NWIN + 1, dtype=jnp.int32) * WN
                           ).astype(jnp.int32)
    cnt = bnd[1:] - bnd[:-1]
    pcnt = ((cnt + BE - 1) // BE) * BE
    wlo = jnp.concatenate([jnp.zeros(1, jnp.int32), jnp.cumsum(pcnt)])
    slots = jnp.arange(E3, dtype=jnp.int32)
    wof = jnp.clip(jnp.searchsorted(wlo, slots, side='right').astype(jnp.int32)
                   - 1, 0, NWIN - 1)
    off = slots - wlo[wof]
    spos = jnp.clip(bnd[wof] + off, 0, E2 - 1)
    valid = off < cnt[wof]
    src_pad = jnp.where(valid, ssrc[spos], N).astype(jnp.int32)
    dst_pad = jnp.where(valid, sdst[spos], wof * WN).astype(jnp.int32)
    win_id = wof[::BE]
    first = jnp.concatenate(
        [jnp.ones(1, jnp.int32),
         (win_id[1:] != win_id[:-1]).astype(jnp.int32)])
    dst3d = dst_pad.reshape(NBLK, 1, BE)

    xp = jnp.pad(x, ((0, NB - N), (0, 0)))

    Wc = W_in @ gat1_W
    bc = (b_in @ gat1_W).reshape(1, -1)
    as1 = gat1_att_src.reshape(1, -1)
    ad1 = gat1_att_dst.reshape(1, -1)
    T1, asrc1, adst1, amax1 = _dense(xp, Wc, bc, as1, ad1, 4, 64)
    G1 = jnp.concatenate(
        [jnp.concatenate([T1[:N], asrc1[:N], jnp.zeros((N, 124))], axis=1),
         jnp.full((1, 384), -1e30, jnp.float32)], axis=0)
    adstw1 = jnp.pad(adst1[:N], ((0, NPAD - N), (0, 0)))
    R1 = _sc_gather(G1, src_pad, E3, 384)
    agg1 = _edge(R1, adstw1, amax1, dst3d, win_id, first, 4, 64, NPAD)
    h1 = _norm(agg1, gat1_bias.reshape(1, -1), 4, 64)

    as2 = gat2_att_src.reshape(1, -1)
    ad2 = gat2_att_dst.reshape(1, -1)
    zb2 = jnp.zeros((1, 64), jnp.float32)
    T2, asrc2, adst2, amax2 = _dense(h1, gat2_W, zb2, as2, ad2, 1, 64)
    G2 = jnp.concatenate(
        [jnp.concatenate([T2[:N], asrc2[:N], jnp.zeros((N, 63))], axis=1),
         jnp.full((1, 128), -1e30, jnp.float32)], axis=0)
    adstw2 = adst2[:NPAD]
    R2 = _sc_gather(G2, src_pad, E3, 128)
    agg2 = _edge(R2, adstw2, amax2, dst3d, win_id, first, 1, 64, NPAD)
    h2 = _norm(agg2, gat2_bias.reshape(1, -1), 1, 64)

    h2p = jnp.pad(h2[:N], ((0, NB - N), (0, 0)))
    scores, _acc, out10 = _head(
        h2p, fa_W1, fa_b1.reshape(1, -1), fa_W2, fa_b2.reshape(1, -1),
        cls_W, cls_b.reshape(1, -1), float(N))
    return (out10, scores[:N, 0])
